# baseline (speedup 1.0000x reference)
import functools

import jax
import jax.numpy as jnp
from jax import lax
from jax.experimental import pallas as pl
from jax.experimental.pallas import tpu as pltpu
from jax.experimental.pallas import tpu_sc as plsc

_NC = 2
_NS = 16
_NW = _NC * _NS

_LANES = 16
_NUM_LM = 68
_LM_PAD = 80
_N_CHUNKS = _LM_PAD // _LANES
_NB = 128
_JH_CHUNKS = 3
_LV = 2 * _NUM_LM

_GATHER_DNUMS = lax.GatherDimensionNumbers(
    offset_dims=(), collapsed_slice_dims=(0,), start_index_map=(0,))


def _splat(vec, idx_scalar):
    rr = jnp.full((_LANES, 1), idx_scalar, jnp.int32)
    return lax.gather(vec, rr, _GATHER_DNUMS, slice_sizes=(1,),
                      mode=lax.GatherScatterMode.PROMISE_IN_BOUNDS)


def _loss_kernel(n, v, m2_hbm, lm_hbm, lv_hbm, out_hbm,
                 lm_v, lv_v, idx0_v, idx1_v, idx2_v, v0_v, v1_v, v2_v,
                 acc_v, sem, sem_in):
    idx_refs = (idx0_v, idx1_v, idx2_v)
    val_refs = (v0_v, v1_v, v2_v)
    wid = lax.axis_index("s") * _NC + lax.axis_index("c")
    nt = wid // 4
    ch = (wid // 2) % 2
    jh = wid % 2
    is_h1 = jh == 1

    cp_lm = pltpu.async_copy(lm_hbm, lm_v.at[pl.ds(0, _NUM_LM)], sem_in)
    cp_lv = pltpu.async_copy(lv_hbm, lv_v, sem_in)

    lane = lax.iota(jnp.int32, _LANES)
    rows_per_ch = (v // 8) * 64

    def chunk_pair(ref, q):
        a = ref[pl.ds(q * _LANES, _LANES)]
        b = ref[pl.ds(min(q + _JH_CHUNKS, _N_CHUNKS - 1) * _LANES, _LANES)]
        return jnp.where(is_h1, b, a)

    cp_lm.wait()
    for q in range(_JH_CHUNKS):
        lmj = chunk_pair(lm_v, q)
        r_idx = ch * rows_per_ch + (lmj >> 3) * 64 + nt * 8 + (lmj & 7)
        valid = jh * _JH_CHUNKS * _LANES + q * _LANES + lane < _NUM_LM
        idx_refs[q][...] = jnp.where(valid, r_idx, 0)
        pltpu.async_copy(m2_hbm.at[idx_refs[q]], val_refs[q], sem)

    zeros = jnp.zeros((_LANES,), jnp.float32)
    tail_valid = _NUM_LM - _JH_CHUNKS * _LANES
    acc_a = zeros
    acc_b = zeros
    cp_lv.wait()
    for g in range(_JH_CHUNKS):
        pltpu.make_async_copy(m2_hbm.at[idx_refs[g]], val_refs[g],
                              sem).wait()
        base0 = 32 * g
        if g <= 1:
            base1 = min(96 + 32 * g, _LV - _LANES)
            shift1 = 96 + 32 * g - base1
        else:
            base1, shift1 = base0, 0
        wA0 = lv_v[pl.ds(base0, _LANES)]
        wA1 = lv_v[pl.ds(base0 + _LANES, _LANES)]
        wB0 = lv_v[pl.ds(base1, _LANES)]
        wB1 = lv_v[pl.ds(min(base1 + _LANES, _LV - _LANES), _LANES)]
        w0 = jnp.where(is_h1, wB0, wA0)
        w1 = jnp.where(is_h1, wB1, wA1)
        shift = jnp.where(is_h1, shift1, 0)

        vals = val_refs[g]
        for r in range(_LANES):
            ln = jnp.minimum(2 * r + ch + shift - (0 if r < 8 else 16), 15)
            t = _splat(w0 if r < 8 else w1, ln)
            racc = zeros
            for k in range(_NB // _LANES):
                d = vals[r, pl.ds(k * _LANES, _LANES)] - t
                racc = racc + d * d
            if g * _LANES + r < tail_valid:
                acc_a = acc_a + racc
            else:
                acc_b = acc_b + racc

    acc_v[...] = acc_a + jnp.where(is_h1, zeros, acc_b)
    pltpu.sync_copy(acc_v, out_hbm.at[wid])


def kernel(mapping, lm, landmarks):
    n, two, v = mapping.shape

    m2 = (mapping
          .reshape(n // _NB, _NB, two, v // 8, 8)
          .transpose(2, 3, 0, 4, 1)
          .reshape(two * (v // 8) * (n // _NB) * 8, _NB))

    lm32 = lm.astype(jnp.int32)
    lv = landmarks.reshape(_LV)

    mesh = plsc.VectorSubcoreMesh(core_axis_name="c", subcore_axis_name="s",
                                  num_cores=_NC, num_subcores=_NS)
    partials = pl.kernel(
        functools.partial(_loss_kernel, n, v),
        out_type=jax.ShapeDtypeStruct((_NW, _LANES), jnp.float32),
        mesh=mesh,
        scratch_types=[
            pltpu.VMEM((_LM_PAD,), jnp.int32),
            pltpu.VMEM((_LV,), jnp.float32),
            pltpu.VMEM((_LANES,), jnp.int32),
            pltpu.VMEM((_LANES,), jnp.int32),
            pltpu.VMEM((_LANES,), jnp.int32),
            pltpu.VMEM((_LANES, _NB), jnp.float32),
            pltpu.VMEM((_LANES, _NB), jnp.float32),
            pltpu.VMEM((_LANES, _NB), jnp.float32),
            pltpu.VMEM((_LANES,), jnp.float32),
            pltpu.SemaphoreType.DMA,
            pltpu.SemaphoreType.DMA,
        ],
    )(m2, lm32, lv)
    return jnp.sum(partials) / n


# speedup vs baseline: 1.5443x; 1.5443x over previous
import functools

import jax
import jax.numpy as jnp
from jax import lax
from jax.experimental import pallas as pl
from jax.experimental.pallas import tpu as pltpu
from jax.experimental.pallas import tpu_sc as plsc

_NC = 2
_NS = 16
_NW = _NC * _NS

_LANES = 16
_NUM_LM = 68
_LM_PAD = 80
_N_CHUNKS = _LM_PAD // _LANES
_NB = 128
_JH_CHUNKS = 3
_PK = 2 * _LM_PAD

_GATHER_DNUMS = lax.GatherDimensionNumbers(
    offset_dims=(), collapsed_slice_dims=(0,), start_index_map=(0,))


def _splat(vec, idx_scalar):
    rr = jnp.full((_LANES, 1), idx_scalar, jnp.int32)
    return lax.gather(vec, rr, _GATHER_DNUMS, slice_sizes=(1,),
                      mode=lax.GatherScatterMode.PROMISE_IN_BOUNDS)


def _loss_kernel(n, v, m2_hbm, lm_hbm, pk_hbm, out_hbm,
                 lm_v, pk_v, idx0_v, idx1_v, idx2_v, v0_v, v1_v, v2_v,
                 acc_v, sem):
    idx_refs = (idx0_v, idx1_v, idx2_v)
    val_refs = (v0_v, v1_v, v2_v)
    wid = lax.axis_index("s") * _NC + lax.axis_index("c")
    nt = wid // 4
    ch = (wid // 2) % 2
    jh = wid % 2
    is_c1 = ch == 1
    is_h1 = jh == 1

    pltpu.sync_copy(lm_hbm, lm_v)
    pltpu.sync_copy(pk_hbm, pk_v)

    rows_per_ch = (v // 8) * 64

    def chunk_pair(ref, base, q):
        a = ref[pl.ds(base + q * _LANES, _LANES)]
        b = ref[pl.ds(base + min(q + _JH_CHUNKS, _N_CHUNKS - 1) * _LANES,
                      _LANES)]
        return jnp.where(is_h1, b, a)

    for q in range(_JH_CHUNKS):
        lmj = chunk_pair(lm_v, 0, q)
        idx_refs[q][...] = (ch * rows_per_ch + (lmj >> 3) * 64
                            + nt * 8 + (lmj & 7))
        pltpu.async_copy(m2_hbm.at[idx_refs[q]], val_refs[q], sem)

    zeros = jnp.zeros((_LANES,), jnp.float32)
    tail_valid = _NUM_LM - _JH_CHUNKS * _LANES
    acc_a = zeros
    acc_b = zeros
    for g in range(_JH_CHUNKS):
        pltpu.make_async_copy(m2_hbm.at[idx_refs[g]], val_refs[g],
                              sem).wait()
        tvec = jnp.where(is_c1, chunk_pair(pk_v, _LM_PAD, g),
                         chunk_pair(pk_v, 0, g))
        vals = val_refs[g]
        for r in range(_LANES):
            t = _splat(tvec, r)
            racc = zeros
            for k in range(_NB // _LANES):
                d = vals[r, pl.ds(k * _LANES, _LANES)] - t
                racc = racc + d * d
            if g * _LANES + r < tail_valid:
                acc_a = acc_a + racc
            else:
                acc_b = acc_b + racc

    acc_v[...] = acc_a + jnp.where(is_h1, zeros, acc_b)
    pltpu.sync_copy(acc_v, out_hbm.at[wid])


def kernel(mapping, lm, landmarks):
    n, two, v = mapping.shape
    num_lm = lm.shape[0]

    m2 = (mapping
          .reshape(n // _NB, _NB, two, v // 8, 8)
          .transpose(2, 3, 0, 4, 1)
          .reshape(two * (v // 8) * (n // _NB) * 8, _NB))

    lm_pad = jnp.zeros((_LM_PAD,), jnp.int32).at[:num_lm].set(
        lm.astype(jnp.int32))
    pk = jnp.zeros((2, _LM_PAD), jnp.float32)
    pk = pk.at[0, :num_lm].set(landmarks[:, 0])
    pk = pk.at[1, :num_lm].set(landmarks[:, 1])
    pk = pk.reshape(_PK)

    mesh = plsc.VectorSubcoreMesh(core_axis_name="c", subcore_axis_name="s",
                                  num_cores=_NC, num_subcores=_NS)
    partials = pl.kernel(
        functools.partial(_loss_kernel, n, v),
        out_type=jax.ShapeDtypeStruct((_NW, _LANES), jnp.float32),
        mesh=mesh,
        scratch_types=[
            pltpu.VMEM((_LM_PAD,), jnp.int32),
            pltpu.VMEM((_PK,), jnp.float32),
            pltpu.VMEM((_LANES,), jnp.int32),
            pltpu.VMEM((_LANES,), jnp.int32),
            pltpu.VMEM((_LANES,), jnp.int32),
            pltpu.VMEM((_LANES, _NB), jnp.float32),
            pltpu.VMEM((_LANES, _NB), jnp.float32),
            pltpu.VMEM((_LANES, _NB), jnp.float32),
            pltpu.VMEM((_LANES,), jnp.float32),
            pltpu.SemaphoreType.DMA,
        ],
    )(m2, lm_pad, pk)
    return jnp.sum(partials) / n
